# baseline (speedup 1.0000x reference)
import functools

import jax
import jax.numpy as jnp
from jax import lax
from jax.experimental import pallas as pl
from jax.experimental.pallas import tpu as pltpu
from jax.experimental.pallas import tpu_sc as plsc

N = 10000
F = 128
H = 128
E = 320000

NC = 2
NS = 16
NW = NC * NS
CHUNK = 128
E_PAD = 327680
NCHUNK = E_PAD // CHUNK
CPW = NCHUNK // NW
CPT = NCHUNK // NS
NRB = 4
NIB = 8
HH = H // 2
AGG_ROWS = 10112
RPT = AGG_ROWS // NS
LAST = N - 15 * RPT

_mesh = plsc.VectorSubcoreMesh(
    core_axis_name="c", subcore_axis_name="s", num_cores=NC, num_subcores=NS)


@functools.partial(
    pl.kernel,
    out_type=jax.ShapeDtypeStruct((NC, AGG_ROWS, 16), jnp.float32),
    mesh=_mesh,
    scratch_types=[
        pltpu.VMEM_SHARED((AGG_ROWS, 16), jnp.float32),
        pltpu.VMEM((CPW, CHUNK), jnp.int32),
        pltpu.VMEM((CHUNK, 16), jnp.float32),
        pltpu.SemaphoreType.DMA,
    ],
    compiler_params=pltpu.CompilerParams(use_tc_tiling_on_sc=False),
)
def _deg_kernel(ei, ones16, zeros16, out, degm, idx_all, ones_v, sem):
    c = lax.axis_index("c")
    s = lax.axis_index("s")
    wid = c * NS + s
    base = wid * CPW
    pltpu.sync_copy(zeros16.at[pl.ds(s * RPT, RPT)], degm.at[pl.ds(s * RPT, RPT)])
    pltpu.sync_copy(ones16, ones_v)
    pltpu.sync_copy(ei.at[1, pl.ds(base, CPW)], idx_all)
    plsc.subcore_barrier()

    for g in range(0, CPW, 16):
        for b in range(16):
            pltpu.async_copy(
                ones_v, degm.at[idx_all.at[g + b]], sem, add=True)
        for b in range(16):
            pltpu.make_async_copy(ones_v, degm.at[idx_all.at[0]], sem).wait()

    plsc.subcore_barrier()
    pltpu.sync_copy(degm.at[pl.ds(s * RPT, RPT)], out.at[c, pl.ds(s * RPT, RPT)])


@functools.partial(
    pl.kernel,
    out_type=jax.ShapeDtypeStruct((NC, AGG_ROWS, HH), jnp.float32),
    mesh=_mesh,
    scratch_types=[
        pltpu.VMEM_SHARED((AGG_ROWS, HH), jnp.float32),
        pltpu.VMEM_SHARED((AGG_ROWS, HH), jnp.float32),
        pltpu.VMEM((NIB, 2, CHUNK), jnp.int32),
        pltpu.VMEM((NRB, CHUNK, HH), jnp.float32),
        [pltpu.SemaphoreType.DMA] * NIB,
        [pltpu.SemaphoreType.DMA] * NRB,
        [pltpu.SemaphoreType.DMA] * NRB,
    ],
    compiler_params=pltpu.CompilerParams(use_tc_tiling_on_sc=False),
)
def _agg_kernel(xa, xb, ei, zer, out, agg, xs, eb, rows, isems, gsems, ssems):
    c = lax.axis_index("c")
    s = lax.axis_index("s")
    base = s * CPT
    r0 = s * RPT
    pltpu.sync_copy(zer.at[pl.ds(r0, RPT)], agg.at[pl.ds(r0, RPT)])

    @pl.when(jnp.logical_and(c == 0, s < 15))
    def _():
        pltpu.sync_copy(xa.at[pl.ds(r0, RPT)], xs.at[pl.ds(r0, RPT)])

    @pl.when(jnp.logical_and(c == 1, s < 15))
    def _():
        pltpu.sync_copy(xb.at[pl.ds(r0, RPT)], xs.at[pl.ds(r0, RPT)])

    @pl.when(jnp.logical_and(c == 0, s == 15))
    def _():
        pltpu.sync_copy(xa.at[pl.ds(15 * RPT, LAST)], xs.at[pl.ds(15 * RPT, LAST)])
        pltpu.sync_copy(zer.at[pl.ds(N, 8)], xs.at[pl.ds(N, 8)])

    @pl.when(jnp.logical_and(c == 1, s == 15))
    def _():
        pltpu.sync_copy(xb.at[pl.ds(15 * RPT, LAST)], xs.at[pl.ds(15 * RPT, LAST)])
        pltpu.sync_copy(zer.at[pl.ds(N, 8)], xs.at[pl.ds(N, 8)])

    plsc.subcore_barrier()

    def fetch_idx(j, ib):
        pltpu.async_copy(ei.at[0, base + j], eb.at[ib, 0], isems[ib])
        pltpu.async_copy(ei.at[1, base + j], eb.at[ib, 1], isems[ib])

    def wait_idx(ib):
        pltpu.make_async_copy(ei.at[0, base], eb.at[ib, 0], isems[ib]).wait()
        pltpu.make_async_copy(ei.at[0, base], eb.at[ib, 1], isems[ib]).wait()

    def start_gather(ib, rb):
        pltpu.async_copy(xs.at[eb.at[ib, 0]], rows.at[rb], gsems[rb])

    def wait_gather(rb):
        pltpu.make_async_copy(
            xs.at[eb.at[0, 0]], rows.at[rb], gsems[rb]).wait()

    def wait_scatter(rb):
        pltpu.make_async_copy(
            rows.at[rb], agg.at[eb.at[0, 1]], ssems[rb]).wait()

    for j0 in range(NRB):
        fetch_idx(j0, j0)
    for j0 in range(3):
        wait_idx(j0)
        start_gather(j0, j0)

    @pl.loop(0, CPT, step=NIB)
    def _step(j):
        for b in range(NIB):
            jj = j + b
            rb = b % NRB
            ib = b % NIB
            wait_gather(rb)
            pltpu.async_copy(rows.at[rb], agg.at[eb.at[ib, 1]], ssems[rb],
                             add=True)

            @pl.when(jj + NRB < CPT)
            def _fetch():
                fetch_idx(jj + NRB, (ib + NRB) % NIB)

            @pl.when(jj + 3 < CPT)
            def _next_gather():
                @pl.when(jj >= 1)
                def _wait_prev_scatter():
                    wait_scatter((rb + 3) % NRB)

                wait_idx((ib + 3) % NIB)
                start_gather((ib + 3) % NIB, (rb + 3) % NRB)

    for k in range(NRB):
        wait_scatter(k)
    plsc.subcore_barrier()
    pltpu.sync_copy(agg.at[pl.ds(r0, RPT)], out.at[c, pl.ds(r0, RPT)])


def _mm_body(x_ref, h_ref, w1_ref, w2_ref, xw_ref):
    xw_ref[...] = (
        jnp.dot(x_ref[...], w1_ref[...], preferred_element_type=jnp.float32)
        + jnp.dot(h_ref[...], w2_ref[...], preferred_element_type=jnp.float32))


def _mm_call(x, h, w1, w2):
    blk = 1000
    grid = N // blk
    return pl.pallas_call(
        _mm_body,
        grid=(grid,),
        in_specs=[
            pl.BlockSpec((blk, F), lambda i: (i, 0)),
            pl.BlockSpec((blk, H), lambda i: (i, 0)),
            pl.BlockSpec((F, H), lambda i: (0, 0)),
            pl.BlockSpec((H, H), lambda i: (0, 0)),
        ],
        out_specs=pl.BlockSpec((blk, H), lambda i: (i, 0)),
        out_shape=jax.ShapeDtypeStruct((N, H), jnp.float32),
    )(x, h, w1, w2)


def _scale_body(xw_ref, d0_ref, d1_ref, xa_ref, xb_ref, dinv_ref):
    dinv = lax.rsqrt(d0_ref[0][:, 0:1] + d1_ref[0][:, 0:1] + 1.0)
    dinv_ref[...] = dinv
    xws = xw_ref[...] * dinv
    xa_ref[...] = xws[:, :HH]
    xb_ref[...] = xws[:, HH:]


def _scale_call(xw, deg):
    blk = 1000
    grid = N // blk
    return pl.pallas_call(
        _scale_body,
        grid=(grid,),
        in_specs=[
            pl.BlockSpec((blk, H), lambda i: (i, 0)),
            pl.BlockSpec((1, blk, 16), lambda i: (0, i, 0)),
            pl.BlockSpec((1, blk, 16), lambda i: (1, i, 0)),
        ],
        out_specs=[
            pl.BlockSpec((blk, HH), lambda i: (i, 0)),
            pl.BlockSpec((blk, HH), lambda i: (i, 0)),
            pl.BlockSpec((blk, 1), lambda i: (i, 0)),
        ],
        out_shape=[
            jax.ShapeDtypeStruct((N, HH), jnp.float32),
            jax.ShapeDtypeStruct((N, HH), jnp.float32),
            jax.ShapeDtypeStruct((N, 1), jnp.float32),
        ],
    )(xw, deg, deg)


def _post_body(a0_ref, a1_ref, xa_ref, xb_ref, dinv_ref, bm_ref, cm_ref,
               gb_ref, v_ref, o_ref, nh_ref):
    a = jnp.concatenate([a0_ref[0], a1_ref[0]], axis=-1)
    xws = jnp.concatenate([xa_ref[...], xb_ref[...]], axis=-1)
    conv = (a + xws) * dinv_ref[...] + gb_ref[...]
    nh = jax.nn.sigmoid(bm_ref[...] + conv)
    nh_ref[...] = nh
    o_ref[...] = cm_ref[...] + jnp.dot(nh, v_ref[...],
                                       preferred_element_type=jnp.float32)


def _post_call(aggs, xa, xb, dinv, bm, cm, gb, v):
    blk = 1000
    grid = N // blk
    return pl.pallas_call(
        _post_body,
        grid=(grid,),
        in_specs=[
            pl.BlockSpec((1, blk, HH), lambda i: (0, i, 0)),
            pl.BlockSpec((1, blk, HH), lambda i: (1, i, 0)),
            pl.BlockSpec((blk, HH), lambda i: (i, 0)),
            pl.BlockSpec((blk, HH), lambda i: (i, 0)),
            pl.BlockSpec((blk, 1), lambda i: (i, 0)),
            pl.BlockSpec((blk, H), lambda i: (i, 0)),
            pl.BlockSpec((blk, F), lambda i: (i, 0)),
            pl.BlockSpec((1, H), lambda i: (0, 0)),
            pl.BlockSpec((H, F), lambda i: (0, 0)),
        ],
        out_specs=[
            pl.BlockSpec((blk, F), lambda i: (i, 0)),
            pl.BlockSpec((blk, H), lambda i: (i, 0)),
        ],
        out_shape=[
            jax.ShapeDtypeStruct((N, F), jnp.float32),
            jax.ShapeDtypeStruct((N, H), jnp.float32),
        ],
    )(aggs, aggs, xa, xb, dinv, bm, cm, gb, v)


def kernel(x, hidden_state, edge_index, gcn_W, gcn_b, b_matrix, v_matrix,
           c_matrix):
    ei = jnp.pad(edge_index, ((0, 0), (0, E_PAD - E)),
                 constant_values=N).reshape(2, NCHUNK, CHUNK)

    ones16 = jnp.ones((CHUNK, 16), jnp.float32)
    zeros16 = jnp.zeros((AGG_ROWS, 16), jnp.float32)
    zer = jnp.zeros((AGG_ROWS, HH), jnp.float32)

    xw = _mm_call(x, hidden_state, gcn_W[:F], gcn_W[F:])
    deg = _deg_kernel(ei, ones16, zeros16)
    xa, xb, dinv = _scale_call(xw, deg)

    aggs = _agg_kernel(xa, xb, ei, zer)

    o, nh = _post_call(aggs, xa, xb, dinv, b_matrix, c_matrix,
                       gcn_b.reshape(1, H), v_matrix)
    return (o, nh)


# speedup vs baseline: 1.0002x; 1.0002x over previous
import functools

import jax
import jax.numpy as jnp
from jax import lax
from jax.experimental import pallas as pl
from jax.experimental.pallas import tpu as pltpu
from jax.experimental.pallas import tpu_sc as plsc

N = 10000
F = 128
H = 128
E = 320000

NC = 2
NS = 16
NW = NC * NS
CHUNK = 128
E_PAD = 327680
NCHUNK = E_PAD // CHUNK
CPW = NCHUNK // NW
CPT = NCHUNK // NS
NRB = 5
NIB = 10
GL = NRB - 1
FL = NRB
HH = H // 2
AGG_ROWS = 10112
RPT = AGG_ROWS // NS
LAST = N - 15 * RPT

_mesh = plsc.VectorSubcoreMesh(
    core_axis_name="c", subcore_axis_name="s", num_cores=NC, num_subcores=NS)


@functools.partial(
    pl.kernel,
    out_type=jax.ShapeDtypeStruct((NC, AGG_ROWS, 16), jnp.float32),
    mesh=_mesh,
    scratch_types=[
        pltpu.VMEM_SHARED((AGG_ROWS, 16), jnp.float32),
        pltpu.VMEM((CPW, CHUNK), jnp.int32),
        pltpu.VMEM((CHUNK, 16), jnp.float32),
        pltpu.SemaphoreType.DMA,
    ],
    compiler_params=pltpu.CompilerParams(use_tc_tiling_on_sc=False),
)
def _deg_kernel(ei, ones16, zeros16, out, degm, idx_all, ones_v, sem):
    c = lax.axis_index("c")
    s = lax.axis_index("s")
    wid = c * NS + s
    base = wid * CPW
    pltpu.sync_copy(zeros16.at[pl.ds(s * RPT, RPT)], degm.at[pl.ds(s * RPT, RPT)])
    pltpu.sync_copy(ones16, ones_v)
    pltpu.sync_copy(ei.at[1, pl.ds(base, CPW)], idx_all)
    plsc.subcore_barrier()

    for g in range(0, CPW, 16):
        for b in range(16):
            pltpu.async_copy(
                ones_v, degm.at[idx_all.at[g + b]], sem, add=True)
        for b in range(16):
            pltpu.make_async_copy(ones_v, degm.at[idx_all.at[0]], sem).wait()

    plsc.subcore_barrier()
    pltpu.sync_copy(degm.at[pl.ds(s * RPT, RPT)], out.at[c, pl.ds(s * RPT, RPT)])


@functools.partial(
    pl.kernel,
    out_type=jax.ShapeDtypeStruct((NC, AGG_ROWS, HH), jnp.float32),
    mesh=_mesh,
    scratch_types=[
        pltpu.VMEM_SHARED((AGG_ROWS, HH), jnp.float32),
        pltpu.VMEM_SHARED((AGG_ROWS, HH), jnp.float32),
        pltpu.VMEM((NIB, 2, CHUNK), jnp.int32),
        pltpu.VMEM((NRB, CHUNK, HH), jnp.float32),
        [pltpu.SemaphoreType.DMA] * NIB,
        [pltpu.SemaphoreType.DMA] * NRB,
        [pltpu.SemaphoreType.DMA] * NRB,
    ],
    compiler_params=pltpu.CompilerParams(use_tc_tiling_on_sc=False),
)
def _agg_kernel(xa, xb, ei, zer, out, agg, xs, eb, rows, isems, gsems, ssems):
    c = lax.axis_index("c")
    s = lax.axis_index("s")
    base = s * CPT
    r0 = s * RPT
    pltpu.sync_copy(zer.at[pl.ds(r0, RPT)], agg.at[pl.ds(r0, RPT)])

    @pl.when(jnp.logical_and(c == 0, s < 15))
    def _():
        pltpu.sync_copy(xa.at[pl.ds(r0, RPT)], xs.at[pl.ds(r0, RPT)])

    @pl.when(jnp.logical_and(c == 1, s < 15))
    def _():
        pltpu.sync_copy(xb.at[pl.ds(r0, RPT)], xs.at[pl.ds(r0, RPT)])

    @pl.when(jnp.logical_and(c == 0, s == 15))
    def _():
        pltpu.sync_copy(xa.at[pl.ds(15 * RPT, LAST)], xs.at[pl.ds(15 * RPT, LAST)])
        pltpu.sync_copy(zer.at[pl.ds(N, 8)], xs.at[pl.ds(N, 8)])

    @pl.when(jnp.logical_and(c == 1, s == 15))
    def _():
        pltpu.sync_copy(xb.at[pl.ds(15 * RPT, LAST)], xs.at[pl.ds(15 * RPT, LAST)])
        pltpu.sync_copy(zer.at[pl.ds(N, 8)], xs.at[pl.ds(N, 8)])

    plsc.subcore_barrier()

    def fetch_idx(j, ib):
        pltpu.async_copy(ei.at[0, base + j], eb.at[ib, 0], isems[ib])
        pltpu.async_copy(ei.at[1, base + j], eb.at[ib, 1], isems[ib])

    def wait_idx(ib):
        pltpu.make_async_copy(ei.at[0, base], eb.at[ib, 0], isems[ib]).wait()
        pltpu.make_async_copy(ei.at[0, base], eb.at[ib, 1], isems[ib]).wait()

    def start_gather(ib, rb):
        pltpu.async_copy(xs.at[eb.at[ib, 0]], rows.at[rb], gsems[rb])

    def wait_gather(rb):
        pltpu.make_async_copy(
            xs.at[eb.at[0, 0]], rows.at[rb], gsems[rb]).wait()

    def wait_scatter(rb):
        pltpu.make_async_copy(
            rows.at[rb], agg.at[eb.at[0, 1]], ssems[rb]).wait()

    for j0 in range(FL):
        fetch_idx(j0, j0)
    for j0 in range(GL):
        wait_idx(j0)
        start_gather(j0, j0)

    @pl.loop(0, CPT, step=NIB)
    def _step(j):
        for b in range(NIB):
            jj = j + b
            rb = b % NRB
            ib = b % NIB
            wait_gather(rb)
            pltpu.async_copy(rows.at[rb], agg.at[eb.at[ib, 1]], ssems[rb],
                             add=True)

            @pl.when(jj + FL < CPT)
            def _fetch():
                fetch_idx(jj + FL, (ib + FL) % NIB)

            @pl.when(jj + GL < CPT)
            def _next_gather():
                @pl.when(jj >= 1)
                def _wait_prev_scatter():
                    wait_scatter((rb + GL) % NRB)

                wait_idx((ib + GL) % NIB)
                start_gather((ib + GL) % NIB, (rb + GL) % NRB)

    for k in range(NRB):
        wait_scatter(k)
    plsc.subcore_barrier()
    pltpu.sync_copy(agg.at[pl.ds(r0, RPT)], out.at[c, pl.ds(r0, RPT)])


def _mm_body(x_ref, h_ref, w1_ref, w2_ref, xw_ref):
    xw_ref[...] = (
        jnp.dot(x_ref[...], w1_ref[...], preferred_element_type=jnp.float32)
        + jnp.dot(h_ref[...], w2_ref[...], preferred_element_type=jnp.float32))


def _mm_call(x, h, w1, w2):
    blk = 1000
    grid = N // blk
    return pl.pallas_call(
        _mm_body,
        grid=(grid,),
        in_specs=[
            pl.BlockSpec((blk, F), lambda i: (i, 0)),
            pl.BlockSpec((blk, H), lambda i: (i, 0)),
            pl.BlockSpec((F, H), lambda i: (0, 0)),
            pl.BlockSpec((H, H), lambda i: (0, 0)),
        ],
        out_specs=pl.BlockSpec((blk, H), lambda i: (i, 0)),
        out_shape=jax.ShapeDtypeStruct((N, H), jnp.float32),
    )(x, h, w1, w2)


def _scale_body(xw_ref, d0_ref, d1_ref, xa_ref, xb_ref, dinv_ref):
    dinv = lax.rsqrt(d0_ref[0][:, 0:1] + d1_ref[0][:, 0:1] + 1.0)
    dinv_ref[...] = dinv
    xws = xw_ref[...] * dinv
    xa_ref[...] = xws[:, :HH]
    xb_ref[...] = xws[:, HH:]


def _scale_call(xw, deg):
    blk = 1000
    grid = N // blk
    return pl.pallas_call(
        _scale_body,
        grid=(grid,),
        in_specs=[
            pl.BlockSpec((blk, H), lambda i: (i, 0)),
            pl.BlockSpec((1, blk, 16), lambda i: (0, i, 0)),
            pl.BlockSpec((1, blk, 16), lambda i: (1, i, 0)),
        ],
        out_specs=[
            pl.BlockSpec((blk, HH), lambda i: (i, 0)),
            pl.BlockSpec((blk, HH), lambda i: (i, 0)),
            pl.BlockSpec((blk, 1), lambda i: (i, 0)),
        ],
        out_shape=[
            jax.ShapeDtypeStruct((N, HH), jnp.float32),
            jax.ShapeDtypeStruct((N, HH), jnp.float32),
            jax.ShapeDtypeStruct((N, 1), jnp.float32),
        ],
    )(xw, deg, deg)


def _post_body(a0_ref, a1_ref, xa_ref, xb_ref, dinv_ref, bm_ref, cm_ref,
               gb_ref, v_ref, o_ref, nh_ref):
    a = jnp.concatenate([a0_ref[0], a1_ref[0]], axis=-1)
    xws = jnp.concatenate([xa_ref[...], xb_ref[...]], axis=-1)
    conv = (a + xws) * dinv_ref[...] + gb_ref[...]
    nh = jax.nn.sigmoid(bm_ref[...] + conv)
    nh_ref[...] = nh
    o_ref[...] = cm_ref[...] + jnp.dot(nh, v_ref[...],
                                       preferred_element_type=jnp.float32)


def _post_call(aggs, xa, xb, dinv, bm, cm, gb, v):
    blk = 1000
    grid = N // blk
    return pl.pallas_call(
        _post_body,
        grid=(grid,),
        in_specs=[
            pl.BlockSpec((1, blk, HH), lambda i: (0, i, 0)),
            pl.BlockSpec((1, blk, HH), lambda i: (1, i, 0)),
            pl.BlockSpec((blk, HH), lambda i: (i, 0)),
            pl.BlockSpec((blk, HH), lambda i: (i, 0)),
            pl.BlockSpec((blk, 1), lambda i: (i, 0)),
            pl.BlockSpec((blk, H), lambda i: (i, 0)),
            pl.BlockSpec((blk, F), lambda i: (i, 0)),
            pl.BlockSpec((1, H), lambda i: (0, 0)),
            pl.BlockSpec((H, F), lambda i: (0, 0)),
        ],
        out_specs=[
            pl.BlockSpec((blk, F), lambda i: (i, 0)),
            pl.BlockSpec((blk, H), lambda i: (i, 0)),
        ],
        out_shape=[
            jax.ShapeDtypeStruct((N, F), jnp.float32),
            jax.ShapeDtypeStruct((N, H), jnp.float32),
        ],
    )(aggs, aggs, xa, xb, dinv, bm, cm, gb, v)


def kernel(x, hidden_state, edge_index, gcn_W, gcn_b, b_matrix, v_matrix,
           c_matrix):
    ei = jnp.pad(edge_index, ((0, 0), (0, E_PAD - E)),
                 constant_values=N).reshape(2, NCHUNK, CHUNK)

    ones16 = jnp.ones((CHUNK, 16), jnp.float32)
    zeros16 = jnp.zeros((AGG_ROWS, 16), jnp.float32)
    zer = jnp.zeros((AGG_ROWS, HH), jnp.float32)

    xw = _mm_call(x, hidden_state, gcn_W[:F], gcn_W[F:])
    deg = _deg_kernel(ei, ones16, zeros16)
    xa, xb, dinv = _scale_call(xw, deg)

    aggs = _agg_kernel(xa, xb, ei, zer)

    o, nh = _post_call(aggs, xa, xb, dinv, b_matrix, c_matrix,
                       gcn_b.reshape(1, H), v_matrix)
    return (o, nh)


# speedup vs baseline: 1.0141x; 1.0139x over previous
import functools

import jax
import jax.numpy as jnp
from jax import lax
from jax.experimental import pallas as pl
from jax.experimental.pallas import tpu as pltpu
from jax.experimental.pallas import tpu_sc as plsc

N = 10000
F = 128
H = 128
E = 320000

NC = 2
NS = 16
NW = NC * NS
CHUNK = 128
E_PAD = 327680
NCHUNK = E_PAD // CHUNK
CPW = NCHUNK // NW
CPT = NCHUNK // NS
NRB = 5
NIB = 10
GL = NRB - 1
FL = NRB
HH = H // 2
AGG_ROWS = 10112
RPT = AGG_ROWS // NS
LAST = N - 15 * RPT

_mesh = plsc.VectorSubcoreMesh(
    core_axis_name="c", subcore_axis_name="s", num_cores=NC, num_subcores=NS)


@functools.partial(
    pl.kernel,
    out_type=jax.ShapeDtypeStruct((NC, AGG_ROWS, 8), jnp.float32),
    mesh=_mesh,
    scratch_types=[
        pltpu.VMEM_SHARED((AGG_ROWS, 8), jnp.float32),
        pltpu.VMEM((CPW, CHUNK), jnp.int32),
        pltpu.VMEM((CHUNK, 8), jnp.float32),
        pltpu.SemaphoreType.DMA,
    ],
    compiler_params=pltpu.CompilerParams(use_tc_tiling_on_sc=False),
)
def _deg_kernel(ei, ones16, zeros16, out, degm, idx_all, ones_v, sem):
    c = lax.axis_index("c")
    s = lax.axis_index("s")
    wid = c * NS + s
    base = wid * CPW
    pltpu.sync_copy(zeros16.at[pl.ds(s * RPT, RPT)], degm.at[pl.ds(s * RPT, RPT)])
    pltpu.sync_copy(ones16, ones_v)
    pltpu.sync_copy(ei.at[1, pl.ds(base, CPW)], idx_all)
    plsc.subcore_barrier()

    for g in range(0, CPW, 16):
        for b in range(16):
            pltpu.async_copy(
                ones_v, degm.at[idx_all.at[g + b]], sem, add=True)
        for b in range(16):
            pltpu.make_async_copy(ones_v, degm.at[idx_all.at[0]], sem).wait()

    plsc.subcore_barrier()
    pltpu.sync_copy(degm.at[pl.ds(s * RPT, RPT)], out.at[c, pl.ds(s * RPT, RPT)])


@functools.partial(
    pl.kernel,
    out_type=jax.ShapeDtypeStruct((NC, AGG_ROWS, HH), jnp.float32),
    mesh=_mesh,
    scratch_types=[
        pltpu.VMEM_SHARED((AGG_ROWS, HH), jnp.float32),
        pltpu.VMEM_SHARED((AGG_ROWS, HH), jnp.float32),
        pltpu.VMEM((NIB, 2, CHUNK), jnp.int32),
        pltpu.VMEM((NRB, CHUNK, HH), jnp.float32),
        [pltpu.SemaphoreType.DMA] * NIB,
        [pltpu.SemaphoreType.DMA] * NRB,
        [pltpu.SemaphoreType.DMA] * NRB,
    ],
    compiler_params=pltpu.CompilerParams(use_tc_tiling_on_sc=False),
)
def _agg_kernel(xa, xb, ei, zer, out, agg, xs, eb, rows, isems, gsems, ssems):
    c = lax.axis_index("c")
    s = lax.axis_index("s")
    base = s * CPT
    r0 = s * RPT
    pltpu.sync_copy(zer.at[pl.ds(r0, RPT)], agg.at[pl.ds(r0, RPT)])

    @pl.when(jnp.logical_and(c == 0, s < 15))
    def _():
        pltpu.sync_copy(xa.at[pl.ds(r0, RPT)], xs.at[pl.ds(r0, RPT)])

    @pl.when(jnp.logical_and(c == 1, s < 15))
    def _():
        pltpu.sync_copy(xb.at[pl.ds(r0, RPT)], xs.at[pl.ds(r0, RPT)])

    @pl.when(jnp.logical_and(c == 0, s == 15))
    def _():
        pltpu.sync_copy(xa.at[pl.ds(15 * RPT, LAST)], xs.at[pl.ds(15 * RPT, LAST)])
        pltpu.sync_copy(zer.at[pl.ds(N, 8)], xs.at[pl.ds(N, 8)])

    @pl.when(jnp.logical_and(c == 1, s == 15))
    def _():
        pltpu.sync_copy(xb.at[pl.ds(15 * RPT, LAST)], xs.at[pl.ds(15 * RPT, LAST)])
        pltpu.sync_copy(zer.at[pl.ds(N, 8)], xs.at[pl.ds(N, 8)])

    plsc.subcore_barrier()

    def fetch_idx(j, ib):
        pltpu.async_copy(ei.at[0, base + j], eb.at[ib, 0], isems[ib])
        pltpu.async_copy(ei.at[1, base + j], eb.at[ib, 1], isems[ib])

    def wait_idx(ib):
        pltpu.make_async_copy(ei.at[0, base], eb.at[ib, 0], isems[ib]).wait()
        pltpu.make_async_copy(ei.at[0, base], eb.at[ib, 1], isems[ib]).wait()

    def start_gather(ib, rb):
        pltpu.async_copy(xs.at[eb.at[ib, 0]], rows.at[rb], gsems[rb])

    def wait_gather(rb):
        pltpu.make_async_copy(
            xs.at[eb.at[0, 0]], rows.at[rb], gsems[rb]).wait()

    def wait_scatter(rb):
        pltpu.make_async_copy(
            rows.at[rb], agg.at[eb.at[0, 1]], ssems[rb]).wait()

    for j0 in range(FL):
        fetch_idx(j0, j0)
    for j0 in range(GL):
        wait_idx(j0)
        start_gather(j0, j0)

    @pl.loop(0, CPT, step=NIB)
    def _step(j):
        for b in range(NIB):
            jj = j + b
            rb = b % NRB
            ib = b % NIB
            wait_gather(rb)
            pltpu.async_copy(rows.at[rb], agg.at[eb.at[ib, 1]], ssems[rb],
                             add=True)

            @pl.when(jj + FL < CPT)
            def _fetch():
                fetch_idx(jj + FL, (ib + FL) % NIB)

            @pl.when(jj + GL < CPT)
            def _next_gather():
                @pl.when(jj >= 1)
                def _wait_prev_scatter():
                    wait_scatter((rb + GL) % NRB)

                wait_idx((ib + GL) % NIB)
                start_gather((ib + GL) % NIB, (rb + GL) % NRB)

    for k in range(NRB):
        wait_scatter(k)
    plsc.subcore_barrier()
    pltpu.sync_copy(agg.at[pl.ds(r0, RPT)], out.at[c, pl.ds(r0, RPT)])


def _mm_body(x_ref, h_ref, w1_ref, w2_ref, xw_ref):
    xw_ref[...] = (
        jnp.dot(x_ref[...], w1_ref[...], preferred_element_type=jnp.float32)
        + jnp.dot(h_ref[...], w2_ref[...], preferred_element_type=jnp.float32))


def _mm_call(x, h, w1, w2):
    blk = 1000
    grid = N // blk
    return pl.pallas_call(
        _mm_body,
        grid=(grid,),
        in_specs=[
            pl.BlockSpec((blk, F), lambda i: (i, 0)),
            pl.BlockSpec((blk, H), lambda i: (i, 0)),
            pl.BlockSpec((F, H), lambda i: (0, 0)),
            pl.BlockSpec((H, H), lambda i: (0, 0)),
        ],
        out_specs=pl.BlockSpec((blk, H), lambda i: (i, 0)),
        out_shape=jax.ShapeDtypeStruct((N, H), jnp.float32),
    )(x, h, w1, w2)


def _scale_body(xw_ref, d0_ref, d1_ref, xa_ref, xb_ref, dinv_ref):
    dinv = lax.rsqrt(d0_ref[0][:, 0:1] + d1_ref[0][:, 0:1] + 1.0)
    dinv_ref[...] = dinv
    xws = xw_ref[...] * dinv
    xa_ref[...] = xws[:, :HH]
    xb_ref[...] = xws[:, HH:]


def _scale_call(xw, deg):
    blk = 1000
    grid = N // blk
    return pl.pallas_call(
        _scale_body,
        grid=(grid,),
        in_specs=[
            pl.BlockSpec((blk, H), lambda i: (i, 0)),
            pl.BlockSpec((1, blk, 8), lambda i: (0, i, 0)),
            pl.BlockSpec((1, blk, 8), lambda i: (1, i, 0)),
        ],
        out_specs=[
            pl.BlockSpec((blk, HH), lambda i: (i, 0)),
            pl.BlockSpec((blk, HH), lambda i: (i, 0)),
            pl.BlockSpec((blk, 1), lambda i: (i, 0)),
        ],
        out_shape=[
            jax.ShapeDtypeStruct((N, HH), jnp.float32),
            jax.ShapeDtypeStruct((N, HH), jnp.float32),
            jax.ShapeDtypeStruct((N, 1), jnp.float32),
        ],
    )(xw, deg, deg)


def _post_body(a0_ref, a1_ref, xa_ref, xb_ref, dinv_ref, bm_ref, cm_ref,
               gb_ref, v_ref, o_ref, nh_ref):
    a = jnp.concatenate([a0_ref[0], a1_ref[0]], axis=-1)
    xws = jnp.concatenate([xa_ref[...], xb_ref[...]], axis=-1)
    conv = (a + xws) * dinv_ref[...] + gb_ref[...]
    nh = jax.nn.sigmoid(bm_ref[...] + conv)
    nh_ref[...] = nh
    o_ref[...] = cm_ref[...] + jnp.dot(nh, v_ref[...],
                                       preferred_element_type=jnp.float32)


def _post_call(aggs, xa, xb, dinv, bm, cm, gb, v):
    blk = 1000
    grid = N // blk
    return pl.pallas_call(
        _post_body,
        grid=(grid,),
        in_specs=[
            pl.BlockSpec((1, blk, HH), lambda i: (0, i, 0)),
            pl.BlockSpec((1, blk, HH), lambda i: (1, i, 0)),
            pl.BlockSpec((blk, HH), lambda i: (i, 0)),
            pl.BlockSpec((blk, HH), lambda i: (i, 0)),
            pl.BlockSpec((blk, 1), lambda i: (i, 0)),
            pl.BlockSpec((blk, H), lambda i: (i, 0)),
            pl.BlockSpec((blk, F), lambda i: (i, 0)),
            pl.BlockSpec((1, H), lambda i: (0, 0)),
            pl.BlockSpec((H, F), lambda i: (0, 0)),
        ],
        out_specs=[
            pl.BlockSpec((blk, F), lambda i: (i, 0)),
            pl.BlockSpec((blk, H), lambda i: (i, 0)),
        ],
        out_shape=[
            jax.ShapeDtypeStruct((N, F), jnp.float32),
            jax.ShapeDtypeStruct((N, H), jnp.float32),
        ],
    )(aggs, aggs, xa, xb, dinv, bm, cm, gb, v)


def kernel(x, hidden_state, edge_index, gcn_W, gcn_b, b_matrix, v_matrix,
           c_matrix):
    ei = jnp.pad(edge_index, ((0, 0), (0, E_PAD - E)),
                 constant_values=N).reshape(2, NCHUNK, CHUNK)

    ones16 = jnp.ones((CHUNK, 8), jnp.float32)
    zeros16 = jnp.zeros((AGG_ROWS, 8), jnp.float32)
    zer = jnp.zeros((AGG_ROWS, HH), jnp.float32)

    xw = _mm_call(x, hidden_state, gcn_W[:F], gcn_W[F:])
    deg = _deg_kernel(ei, ones16, zeros16)
    xa, xb, dinv = _scale_call(xw, deg)

    aggs = _agg_kernel(xa, xb, ei, zer)

    o, nh = _post_call(aggs, xa, xb, dinv, b_matrix, c_matrix,
                       gcn_b.reshape(1, H), v_matrix)
    return (o, nh)
